# Initial kernel scaffold; baseline (speedup 1.0000x reference)
import jax
import jax.numpy as jnp
from jax.experimental import pallas as pl


def kernel(context, table, W, b):
    raise NotImplementedError("write your pallas kernel here")


# speedup vs baseline: 5.4987x; 5.4987x over previous
import functools

import jax
import jax.numpy as jnp
from jax import lax
from jax.experimental import pallas as pl
from jax.experimental.pallas import tpu as pltpu
from jax.experimental.pallas import tpu_sc as plsc

VOCAB = 1000
EMBED_DIM = 128
REL = 5
LANES = 16
NC = 2
NS = 16
NW = NC * NS


def _project_body(table_ref, w_ref, b_ref, out_ref):
    t = table_ref[...]
    w = w_ref[...]
    p = lax.dot_general(t, w, (((1,), (1,)), ((), ())),
                        preferred_element_type=jnp.float32)
    out_ref[...] = jnp.maximum(p + b_ref[...], 0.0)


def _project(table, W, b):
    return pl.pallas_call(
        _project_body,
        out_shape=jax.ShapeDtypeStruct((VOCAB, REL), jnp.float32),
    )(table, W, b.reshape(1, REL))


def _make_gather(n_total: int, chunk: int):
    assert n_total % (NW * chunk) == 0
    per_w = n_total // NW
    n_chunks = per_w // chunk
    groups = chunk // LANES
    mesh = plsc.VectorSubcoreMesh(core_axis_name="c", subcore_axis_name="s")

    @functools.partial(
        pl.kernel, mesh=mesh,
        out_type=jax.ShapeDtypeStruct((n_total * REL,), jnp.float32),
        compiler_params=pltpu.CompilerParams(needs_layout_passes=False),
        scratch_types=[
            pltpu.VMEM((VOCAB * REL,), jnp.float32),
            pltpu.VMEM((chunk,), jnp.int32),
            pltpu.VMEM((chunk * REL,), jnp.float32),
        ],
    )
    def gather(p_hbm, ctx_hbm, out_hbm, p_v, idx_v, out_v):
        wid = lax.axis_index("s") * NC + lax.axis_index("c")
        base = wid * per_w
        pltpu.sync_copy(p_hbm, p_v)
        iota5 = lax.iota(jnp.int32, LANES) * REL

        def do_chunk(c, carry):
            cbase = base + c * chunk
            pltpu.sync_copy(ctx_hbm.at[pl.ds(cbase, chunk)], idx_v)

            def do_group(g, carry2):
                idx5 = idx_v[pl.ds(g * LANES, LANES)] * REL
                obase = g * (LANES * REL)
                for j in range(REL):
                    vals = plsc.load_gather(p_v, [idx5 + j])
                    plsc.store_scatter(out_v, [iota5 + (obase + j)], vals)
                return carry2

            lax.fori_loop(0, groups, do_group, 0, unroll=4)
            pltpu.sync_copy(out_v, out_hbm.at[pl.ds(cbase * REL, chunk * REL)])
            return carry

        lax.fori_loop(0, n_chunks, do_chunk, 0)

    return gather


def kernel(context, table, W, b):
    B, L = context.shape
    n_total = B * L
    P = _project(table, W, b)
    ctx_flat = context.reshape(n_total)
    out_flat = _make_gather(n_total, 5120)(P.reshape(VOCAB * REL), ctx_flat)
    return out_flat.reshape(B, L, REL)


# speedup vs baseline: 57.7205x; 10.4971x over previous
import functools

import jax
import jax.numpy as jnp
from jax import lax
from jax.experimental import pallas as pl
from jax.experimental.pallas import tpu as pltpu
from jax.experimental.pallas import tpu_sc as plsc

VOCAB = 1000
EMBED_DIM = 128
REL = 5
LANES = 16
NC = 2
NS = 16
NW = NC * NS


def _project_body(w_ref, table_ref, b_ref, out_ref):
    w = w_ref[...]
    t = table_ref[...]
    p = lax.dot_general(w, t, (((1,), (1,)), ((), ())),
                        preferred_element_type=jnp.float32)
    out_ref[...] = jnp.maximum(p + b_ref[...], 0.0)


def _project_t(table, W, b):
    return pl.pallas_call(
        _project_body,
        out_shape=jax.ShapeDtypeStruct((REL, VOCAB), jnp.float32),
    )(W, table, b.reshape(REL, 1))


def _make_gather(L: int, B: int):
    bw = B // NW
    n_lt = L // 8
    mesh = plsc.VectorSubcoreMesh(core_axis_name="c", subcore_axis_name="s")

    @functools.partial(
        pl.kernel, mesh=mesh,
        out_type=jax.ShapeDtypeStruct((REL, L, B), jnp.float32),
        compiler_params=pltpu.CompilerParams(needs_layout_passes=False),
        scratch_types=[
            pltpu.VMEM((REL, VOCAB), jnp.float32),
            pltpu.VMEM((8, bw), jnp.int32),
            pltpu.VMEM((REL, 8, bw), jnp.float32),
        ],
    )
    def gather(pt_hbm, ctx_hbm, out_hbm, pt_v, idx_v, out_v):
        wid = lax.axis_index("s") * NC + lax.axis_index("c")
        b0 = wid * bw
        pltpu.sync_copy(pt_hbm, pt_v)

        def do_chunk(lt, carry):
            r0 = lt * 8
            pltpu.sync_copy(ctx_hbm.at[pl.ds(r0, 8), pl.ds(b0, bw)], idx_v)
            for s in range(8):
                def do_group(g, carry2):
                    k = g * LANES
                    idx = idx_v[s, pl.ds(k, LANES)]
                    for c in range(REL):
                        cvec = jnp.full((LANES,), c, dtype=jnp.int32)
                        out_v[c, s, pl.ds(k, LANES)] = plsc.load_gather(
                            pt_v, [cvec, idx])
                    return carry2

                lax.fori_loop(0, bw // LANES, do_group, 0, unroll=4)
            for c in range(REL):
                pltpu.sync_copy(
                    out_v.at[c],
                    out_hbm.at[c, pl.ds(r0, 8), pl.ds(b0, bw)])
            return carry

        lax.fori_loop(0, n_lt, do_chunk, 0)

    return gather


def kernel(context, table, W, b):
    B, L = context.shape
    Pt = _project_t(table, W, b)
    ctx_t = context.T
    out5 = _make_gather(L, B)(Pt, ctx_t)
    return jnp.transpose(out5, (2, 1, 0))


# speedup vs baseline: 61.1837x; 1.0600x over previous
import functools

import jax
import jax.numpy as jnp
from jax import lax
from jax.experimental import pallas as pl
from jax.experimental.pallas import tpu as pltpu
from jax.experimental.pallas import tpu_sc as plsc

VOCAB = 1000
EMBED_DIM = 128
REL = 5
LANES = 16
NC = 2
NS = 16
NW = NC * NS


def _project_body(w_ref, table_ref, b_ref, out_ref):
    w = w_ref[...]
    t = table_ref[...]
    p = lax.dot_general(w, t, (((1,), (1,)), ((), ())),
                        preferred_element_type=jnp.float32)
    out_ref[...] = jnp.maximum(p + b_ref[...], 0.0)


def _project_t(table, W, b):
    return pl.pallas_call(
        _project_body,
        out_shape=jax.ShapeDtypeStruct((REL, VOCAB), jnp.float32),
    )(W, table, b.reshape(REL, 1))


def _make_gather(L: int, B: int):
    bw = B // NW
    n_lt = L // 8
    mesh = plsc.VectorSubcoreMesh(core_axis_name="c", subcore_axis_name="s")

    @functools.partial(
        pl.kernel, mesh=mesh,
        out_type=jax.ShapeDtypeStruct((REL, L, B), jnp.float32),
        compiler_params=pltpu.CompilerParams(needs_layout_passes=False),
        scratch_types=[
            pltpu.VMEM((REL, VOCAB), jnp.float32),
            pltpu.VMEM((2, 8, bw), jnp.int32),
            pltpu.VMEM((2, REL, 8, bw), jnp.float32),
            pltpu.SemaphoreType.DMA,
            pltpu.SemaphoreType.DMA,
        ],
    )
    def gather(pt_hbm, ctx_hbm, out_hbm, pt_v, idx_v, out_v, sem0, sem1):
        wid = lax.axis_index("s") * NC + lax.axis_index("c")
        b0 = wid * bw
        sems = (sem0, sem1)
        pltpu.sync_copy(pt_hbm, pt_v)

        def out_slices(lt, buf):
            return out_v.at[buf], out_hbm.at[:, pl.ds(lt * 8, 8),
                                             pl.ds(b0, bw)]

        def chunk_body(lt, buf, first):
            pltpu.sync_copy(ctx_hbm.at[pl.ds(lt * 8, 8), pl.ds(b0, bw)],
                            idx_v.at[buf])
            src, dst = out_slices(lt, buf)
            if not first:
                pltpu.make_async_copy(src, dst, sems[buf]).wait()
            for s in range(8):
                def do_group(g, carry):
                    k = g * LANES
                    idx = idx_v[buf, s, pl.ds(k, LANES)]
                    for c in range(REL):
                        cvec = jnp.full((LANES,), c, dtype=jnp.int32)
                        out_v[buf, c, s, pl.ds(k, LANES)] = plsc.load_gather(
                            pt_v, [cvec, idx])
                    return carry

                lax.fori_loop(0, bw // LANES, do_group, 0, unroll=4)
            pltpu.async_copy(src, dst, sems[buf])

        chunk_body(0, 0, True)
        chunk_body(1, 1, True)

        def do_pair(i, carry):
            chunk_body(2 * i, 0, False)
            chunk_body(2 * i + 1, 1, False)
            return carry

        lax.fori_loop(1, (n_lt - 1) // 2, do_pair, 0)
        if n_lt % 2 == 1:
            chunk_body(n_lt - 1, 0, False)
        src, dst = out_slices(0, 1)
        pltpu.make_async_copy(src, dst, sems[1]).wait()
        src, dst = out_slices(0, 0)
        pltpu.make_async_copy(src, dst, sems[0]).wait()

    return gather


def kernel(context, table, W, b):
    B, L = context.shape
    Pt = _project_t(table, W, b)
    ctx_t = context.T
    out5 = _make_gather(L, B)(Pt, ctx_t)
    return jnp.transpose(out5, (2, 1, 0))


# speedup vs baseline: 138.7926x; 2.2685x over previous
import functools

import jax
import jax.numpy as jnp
from jax import lax
from jax.experimental import pallas as pl
from jax.experimental.pallas import tpu as pltpu
from jax.experimental.pallas import tpu_sc as plsc

VOCAB = 1000
EMBED_DIM = 128
REL = 5
LANES = 16
NC = 2
NS = 16
NW = NC * NS


def _project_body(w_ref, table_ref, b_ref, out_ref):
    w = w_ref[...]
    t = table_ref[...]
    p = lax.dot_general(w, t, (((1,), (1,)), ((), ())),
                        preferred_element_type=jnp.float32)
    out_ref[...] = jnp.maximum(p + b_ref[...], 0.0)


def _project_t(table, W, b):
    return pl.pallas_call(
        _project_body,
        out_shape=jax.ShapeDtypeStruct((REL, VOCAB), jnp.float32),
    )(W, table, b.reshape(REL, 1))


def _make_gather(L: int, B: int):
    bw = B // NW
    n_lt = L // 8
    mesh = plsc.VectorSubcoreMesh(core_axis_name="c", subcore_axis_name="s")

    @functools.partial(
        pl.kernel, mesh=mesh,
        out_type=jax.ShapeDtypeStruct((REL, L, B), jnp.float32),
        compiler_params=pltpu.CompilerParams(needs_layout_passes=False),
        scratch_types=[
            pltpu.VMEM((REL * VOCAB,), jnp.float32),
            pltpu.VMEM((2, 8, bw), jnp.int32),
            pltpu.VMEM((2, REL, 8, bw), jnp.float32),
            pltpu.SemaphoreType.DMA,
            pltpu.SemaphoreType.DMA,
        ],
    )
    def gather(pt_hbm, ctx_hbm, out_hbm, pt_v, idx_v, out_v, sem0, sem1):
        wid = lax.axis_index("s") * NC + lax.axis_index("c")
        b0 = wid * bw
        sems = (sem0, sem1)
        pltpu.sync_copy(pt_hbm, pt_v)

        def out_slices(lt, buf):
            return out_v.at[buf], out_hbm.at[:, pl.ds(lt * 8, 8),
                                             pl.ds(b0, bw)]

        def chunk_body(lt, buf, first):
            pltpu.sync_copy(ctx_hbm.at[pl.ds(lt * 8, 8), pl.ds(b0, bw)],
                            idx_v.at[buf])
            src, dst = out_slices(lt, buf)
            if not first:
                pltpu.make_async_copy(src, dst, sems[buf]).wait()
            GB = 4
            for s in range(8):
                def do_group(g, carry):
                    k0 = g * (GB * LANES)
                    idxs = [idx_v[buf, s, pl.ds(k0 + j * LANES, LANES)]
                            for j in range(GB)]
                    vals = [[plsc.load_gather(
                        pt_v.at[pl.ds(c * VOCAB, VOCAB)], [idxs[j]])
                        for c in range(REL)] for j in range(GB)]
                    for j in range(GB):
                        for c in range(REL):
                            out_v[buf, c, s,
                                  pl.ds(k0 + j * LANES, LANES)] = vals[j][c]
                    return carry

                lax.fori_loop(0, bw // (GB * LANES), do_group, 0)
            pltpu.async_copy(src, dst, sems[buf])

        chunk_body(0, 0, True)
        chunk_body(1, 1, True)

        def do_pair(i, carry):
            chunk_body(2 * i, 0, False)
            chunk_body(2 * i + 1, 1, False)
            return carry

        lax.fori_loop(1, (n_lt - 1) // 2, do_pair, 0)
        if n_lt % 2 == 1:
            chunk_body(n_lt - 1, 0, False)
        src, dst = out_slices(0, 1)
        pltpu.make_async_copy(src, dst, sems[1]).wait()
        src, dst = out_slices(0, 0)
        pltpu.make_async_copy(src, dst, sems[0]).wait()

    return gather


def kernel(context, table, W, b):
    B, L = context.shape
    Pt = _project_t(table, W, b)
    ctx_t = context.T
    out5 = _make_gather(L, B)(Pt.reshape(REL * VOCAB), ctx_t)
    return jnp.transpose(out5, (2, 1, 0))


# speedup vs baseline: 180.7038x; 1.3020x over previous
import functools

import jax
import jax.numpy as jnp
from jax import lax
from jax.experimental import pallas as pl
from jax.experimental.pallas import tpu as pltpu
from jax.experimental.pallas import tpu_sc as plsc

VOCAB = 1000
EMBED_DIM = 128
REL = 5
LANES = 16
NC = 2
NS = 16
NW = NC * NS


def _project_body(w_ref, table_ref, b_ref, out_ref):
    w = w_ref[...]
    t = table_ref[...]
    p = lax.dot_general(w, t, (((1,), (1,)), ((), ())),
                        preferred_element_type=jnp.float32)
    out_ref[...] = jnp.maximum(p + b_ref[...], 0.0)


def _project_t(table, W, b):
    return pl.pallas_call(
        _project_body,
        out_shape=jax.ShapeDtypeStruct((REL, VOCAB), jnp.float32),
    )(W, table, b.reshape(REL, 1))


def _make_gather(L: int, B: int):
    bw = B // NW
    n_lt = L // 8
    mesh = plsc.VectorSubcoreMesh(core_axis_name="c", subcore_axis_name="s")

    @functools.partial(
        pl.kernel, mesh=mesh,
        out_type=jax.ShapeDtypeStruct((REL, L, B), jnp.float32),
        compiler_params=pltpu.CompilerParams(needs_layout_passes=False),
        scratch_types=[
            pltpu.VMEM((REL * VOCAB,), jnp.float32),
            pltpu.VMEM((2, 8, bw), jnp.int32),
            pltpu.VMEM((2, REL, 8, bw), jnp.float32),
            pltpu.SemaphoreType.DMA,
            pltpu.SemaphoreType.DMA,
            pltpu.SemaphoreType.DMA,
            pltpu.SemaphoreType.DMA,
        ],
    )
    def gather(pt_hbm, ctx_hbm, out_hbm, pt_v, idx_v, out_v,
               sem0, sem1, isem0, isem1):
        wid = lax.axis_index("s") * NC + lax.axis_index("c")
        b0 = wid * bw
        sems = (sem0, sem1)
        isems = (isem0, isem1)

        def in_slices(lt, buf):
            return ctx_hbm.at[pl.ds(lt * 8, 8), pl.ds(b0, bw)], idx_v.at[buf]

        def out_slices(lt, buf):
            return out_v.at[buf], out_hbm.at[:, pl.ds(lt * 8, 8),
                                             pl.ds(b0, bw)]

        pltpu.async_copy(*in_slices(0, 0), isems[0])
        pltpu.async_copy(*in_slices(1, 1), isems[1])
        pltpu.sync_copy(pt_hbm, pt_v)

        def chunk_body(lt, buf, first, pre_lt=None, pre_guard=None):
            src_i, dst_i = in_slices(lt, buf)
            pltpu.make_async_copy(src_i, dst_i, isems[buf]).wait()
            src, dst = out_slices(lt, buf)
            if not first:
                pltpu.make_async_copy(src, dst, sems[buf]).wait()
            GB = 4
            for s in range(8):
                def do_group(g, carry):
                    k0 = g * (GB * LANES)
                    idxs = [idx_v[buf, s, pl.ds(k0 + j * LANES, LANES)]
                            for j in range(GB)]
                    vals = [[plsc.load_gather(
                        pt_v.at[pl.ds(c * VOCAB, VOCAB)], [idxs[j]])
                        for c in range(REL)] for j in range(GB)]
                    for j in range(GB):
                        for c in range(REL):
                            out_v[buf, c, s,
                                  pl.ds(k0 + j * LANES, LANES)] = vals[j][c]
                    return carry

                lax.fori_loop(0, bw // (GB * LANES), do_group, 0)
            pltpu.async_copy(src, dst, sems[buf])
            if pre_lt is not None:
                def issue():
                    pltpu.async_copy(*in_slices(pre_lt, buf), isems[buf])
                if pre_guard is not None:
                    pl.when(pre_guard)(issue)
                else:
                    issue()

        chunk_body(0, 0, True, pre_lt=2)
        chunk_body(1, 1, True, pre_lt=3)

        def do_pair(i, carry):
            chunk_body(2 * i, 0, False, pre_lt=2 * i + 2)
            chunk_body(2 * i + 1, 1, False, pre_lt=2 * i + 3,
                       pre_guard=2 * i + 3 < n_lt)
            return carry

        lax.fori_loop(1, (n_lt - 1) // 2, do_pair, 0)
        if n_lt % 2 == 1:
            chunk_body(n_lt - 1, 0, False)
        src, dst = out_slices(0, 1)
        pltpu.make_async_copy(src, dst, sems[1]).wait()
        src, dst = out_slices(0, 0)
        pltpu.make_async_copy(src, dst, sems[0]).wait()

    return gather


def kernel(context, table, W, b):
    B, L = context.shape
    Pt = _project_t(table, W, b)
    ctx_t = context.T
    out5 = _make_gather(L, B)(Pt.reshape(REL * VOCAB), ctx_t)
    return jnp.transpose(out5, (2, 1, 0))
